# baseline (speedup 1.0000x reference)
import functools

import jax
import jax.numpy as jnp
from jax import lax
from jax.experimental import pallas as pl
from jax.experimental.pallas import tpu as pltpu
from jax.experimental.pallas import tpu_sc as plsc

D = 1024
B = 4 * 8192
NC, NS = 2, 16
NW = NC * NS
BPW = B // NW
CS = (64, 56)
NBUF = len(CS)
CYC = sum(CS)
RING = BPW // CYC
REM = BPW - RING * CYC
OFF = [sum(CS[:b]) for b in range(NBUF)]


def _gather(table, idx):
  mesh = plsc.VectorSubcoreMesh(core_axis_name="c", subcore_axis_name="s")

  @functools.partial(
      pl.kernel,
      out_type=jax.ShapeDtypeStruct((B, D), jnp.float32),
      mesh=mesh,
      scratch_types=[
          pltpu.VMEM((BPW,), jnp.int32),
          [pltpu.VMEM((c, D), jnp.float32) for c in CS],
          [pltpu.SemaphoreType.DMA for _ in CS],
          [pltpu.SemaphoreType.DMA for _ in CS],
      ],
  )
  def k(table_hbm, idx_hbm, out_hbm, idx_v, rows, si, so):
    wid = lax.axis_index("s") * NC + lax.axis_index("c")
    base = wid * BPW
    pltpu.sync_copy(idx_hbm.at[pl.ds(base, BPW)], idx_v)

    def gather(row, b, n):
      pltpu.async_copy(table_hbm.at[idx_v.at[pl.ds(row, n)]],
                       rows[b].at[pl.ds(0, n)], si[b])

    def put(row, b, n):
      pltpu.async_copy(rows[b].at[pl.ds(0, n)],
                       out_hbm.at[pl.ds(base + row, n)], so[b])

    def wait(b, sem, n):
      pltpu.make_async_copy(table_hbm.at[pl.ds(0, n)],
                            rows[b].at[pl.ds(0, n)], sem[b]).wait()

    for b in range(NBUF):
      gather(OFF[b], b, CS[b])

    @pl.loop(0, RING - 1)
    def _(i):
      row = i * CYC
      for b in range(NBUF):
        wait(b, si, CS[b])
        put(row + OFF[b], b, CS[b])
      for b in range(NBUF):
        wait(b, so, CS[b])
        gather(row + CYC + OFF[b], b, CS[b])

    last = (RING - 1) * CYC
    for b in range(NBUF):
      wait(b, si, CS[b])
      put(last + OFF[b], b, CS[b])

    if REM:
      wait(0, so, CS[0])
      gather(RING * CYC, 0, REM)
      wait(0, si, REM)
      put(RING * CYC, 0, REM)

    wait(0, so, REM if REM else CS[0])
    for b in range(1, NBUF):
      wait(b, so, CS[b])

  return k(table, idx)


@jax.jit
def kernel(x, embedding_weight):
  idx = x.reshape(-1).astype(jnp.int32)
  out = _gather(embedding_weight, idx)
  return out.reshape(x.shape + (D,))


# speedup vs baseline: 1.0384x; 1.0384x over previous
import functools

import jax
import jax.numpy as jnp
from jax import lax
from jax.experimental import pallas as pl
from jax.experimental.pallas import tpu as pltpu
from jax.experimental.pallas import tpu_sc as plsc

D = 1024
B = 4 * 8192
NC, NS = 2, 16
NW = NC * NS
BPW = B // NW
C = 56
NBUF = 2
NFCH = BPW // C
RING = NFCH // NBUF
REM = BPW - NFCH * C


def _gather(table, idx):
  mesh = plsc.VectorSubcoreMesh(core_axis_name="c", subcore_axis_name="s")

  @functools.partial(
      pl.kernel,
      out_type=jax.ShapeDtypeStruct((B, D), jnp.float32),
      mesh=mesh,
      scratch_types=[
          pltpu.VMEM((BPW,), jnp.int32),
          [pltpu.VMEM((C, D), jnp.float32) for _ in range(NBUF)],
          [pltpu.SemaphoreType.DMA for _ in range(NBUF)],
          [pltpu.SemaphoreType.DMA for _ in range(NBUF)],
      ],
  )
  def k(table_hbm, idx_hbm, out_hbm, idx_v, rows, si, so):
    wid = lax.axis_index("s") * NC + lax.axis_index("c")
    base = wid * BPW
    pltpu.sync_copy(idx_hbm.at[pl.ds(base, BPW)], idx_v)

    def gather(row, b, n=C):
      pltpu.async_copy(table_hbm.at[idx_v.at[pl.ds(row, n)]],
                       rows[b].at[pl.ds(0, n)], si[b])

    def put(row, b, n=C):
      pltpu.async_copy(rows[b].at[pl.ds(0, n)],
                       out_hbm.at[pl.ds(base + row, n)], so[b])

    def wait(b, sem, n=C):
      pltpu.make_async_copy(table_hbm.at[pl.ds(0, n)],
                            rows[b].at[pl.ds(0, n)], sem[b]).wait()

    gather(NFCH * C, 1, REM)
    gather(0, 0)
    wait(1, si, REM)
    put(NFCH * C, 1, REM)
    wait(1, so, REM)
    gather(C, 1)

    @pl.loop(0, RING - 1)
    def _(i):
      row = i * NBUF * C
      for b in range(NBUF):
        wait(b, si)
        put(row + b * C, b)
      for b in range(NBUF):
        wait(b, so)
        gather(row + (NBUF + b) * C, b)

    last = (RING - 1) * NBUF * C
    for b in range(NBUF):
      wait(b, si)
      put(last + b * C, b)
    for b in range(NBUF):
      wait(b, so)

  return k(table, idx)


@jax.jit
def kernel(x, embedding_weight):
  idx = x.reshape(-1).astype(jnp.int32)
  out = _gather(embedding_weight, idx)
  return out.reshape(x.shape + (D,))
